# baseline (speedup 1.0000x reference)
import functools

import jax
import jax.numpy as jnp
from jax import lax
from jax.experimental import pallas as pl
from jax.experimental.pallas import tpu as pltpu
from jax.experimental.pallas import tpu_sc as plsc

_CB = 8192
_N4 = 253952
_OFF = (0, 253952, 507904, 753664)


def _pack_stream(x_ref):
    xt = x_ref[...].astype(jnp.bfloat16).T
    lo = lax.bitcast_convert_type(xt[:, :32], jnp.uint16).astype(jnp.uint32)
    hi = lax.bitcast_convert_type(xt[:, 32:], jnp.uint16).astype(jnp.uint32)
    return lax.bitcast_convert_type(lo | (hi << 16), jnp.int32)


def _repack_body(xa_ref, xb_ref, xc_ref, xd_ref, o_ref):
    o_ref[...] = jnp.concatenate(
        [_pack_stream(r) for r in (xa_ref, xb_ref, xc_ref, xd_ref)], axis=1
    )


def _repack(tableT):
    D, V = tableT.shape
    grid = _N4 // _CB
    return pl.pallas_call(
        _repack_body,
        grid=(grid,),
        in_specs=[
            pl.BlockSpec((D, _CB), lambda i, s=s: (0, i + _OFF[s] // _CB))
            for s in range(4)
        ],
        out_specs=pl.BlockSpec((_CB, 2 * D), lambda i: (i, 0)),
        out_shape=jax.ShapeDtypeStruct((_N4, 2 * D), jnp.int32),
    )(tableT, tableT, tableT, tableT)


def _make_sc_gather(D2, B):
    info = plsc.get_sparse_core_info()
    NC, NS = info.num_cores, info.num_subcores
    NW = NC * NS
    assert B % (8 * NW) == 0 and D2 % info.num_lanes == 0
    b_per_w = B // NW
    mesh = plsc.VectorSubcoreMesh(core_axis_name="c", subcore_axis_name="s")

    @functools.partial(
        pl.kernel,
        mesh=mesh,
        out_type=jax.ShapeDtypeStruct((B, D2), jnp.int32),
        scratch_types=[
            pltpu.VMEM((b_per_w // 128, 128), jnp.int32),
            pltpu.VMEM((b_per_w, D2), jnp.int32),
            pltpu.SemaphoreType.DMA,
            pltpu.SemaphoreType.DMA,
        ],
    )
    def gather_k(table_hbm, idx_hbm, out_hbm, idx_v, rows_v, sem_idx, sem):
        wid = lax.axis_index("s") * NC + lax.axis_index("c")
        base = wid * b_per_w
        nj = b_per_w // 128
        for j in range(nj):
            pltpu.async_copy(
                idx_hbm.at[pl.ds(base + j * 128, 128)], idx_v.at[j], sem_idx
            )
        for j in range(nj):
            pltpu.make_async_copy(
                idx_hbm.at[pl.ds(base + j * 128, 128)], idx_v.at[j], sem_idx
            ).wait()
        for j in range(nj):
            pltpu.async_copy(
                table_hbm.at[idx_v.at[j]], rows_v.at[pl.ds(j * 128, 128)], sem
            )
        for j in range(nj):
            pltpu.make_async_copy(
                table_hbm.at[idx_v.at[j]], rows_v.at[pl.ds(j * 128, 128)], sem
            ).wait()
        pltpu.sync_copy(rows_v, out_hbm.at[pl.ds(base, b_per_w)])

    return gather_k


def _mlp_body(g_ref, sel_ref, w1_ref, b1_ref, w2_ref, b2_ref, o_ref):
    sel = sel_ref[...]
    g = g_ref[...]
    w = jnp.where(
        sel < 2,
        jnp.where(sel == 0, g[:, 0:32], g[:, 32:64]),
        jnp.where(sel == 2, g[:, 64:96], g[:, 96:128]),
    )
    wu = lax.bitcast_convert_type(w, jnp.uint32)
    lo = lax.bitcast_convert_type((wu & 0xFFFF).astype(jnp.uint16), jnp.bfloat16)
    hi = lax.bitcast_convert_type((wu >> 16).astype(jnp.uint16), jnp.bfloat16)
    x = jnp.concatenate([lo, hi], axis=1).astype(jnp.float32)
    h = jnp.dot(x, w1_ref[...], preferred_element_type=jnp.float32) + b1_ref[...]
    h = jnp.maximum(h, 0.0)
    y = jnp.dot(h, w2_ref[...], preferred_element_type=jnp.float32) + b2_ref[...]
    ss = jnp.sum(y * y, axis=-1, keepdims=True)
    o_ref[...] = y / jnp.maximum(jnp.sqrt(ss), 1e-12)


def _mlp(gathered, sel, W1, b1, W2, b2, blk=2048):
    B, D2 = gathered.shape
    D = W1.shape[0]
    H = W1.shape[1]
    O = W2.shape[1]
    return pl.pallas_call(
        _mlp_body,
        grid=(B // blk,),
        in_specs=[
            pl.BlockSpec((blk, D2), lambda i: (i, 0)),
            pl.BlockSpec((blk, 1), lambda i: (i, 0)),
            pl.BlockSpec((D, H), lambda i: (0, 0)),
            pl.BlockSpec((1, H), lambda i: (0, 0)),
            pl.BlockSpec((H, O), lambda i: (0, 0)),
            pl.BlockSpec((1, O), lambda i: (0, 0)),
        ],
        out_specs=pl.BlockSpec((blk, O), lambda i: (i, 0)),
        out_shape=jax.ShapeDtypeStruct((B, O), jnp.float32),
    )(gathered, sel, W1, b1.reshape(1, H), W2, b2.reshape(1, O))


def kernel(indices, table, W1, b1, W2, b2):
    idx = indices.astype(jnp.int32)
    B = idx.shape[0]
    V, D = table.shape
    t32 = _repack(table.T)
    s = (
        (idx >= _OFF[1]).astype(jnp.int32)
        + (idx >= _OFF[2]).astype(jnp.int32)
        + (idx >= _OFF[3]).astype(jnp.int32)
    )
    off = jnp.array(_OFF, dtype=jnp.int32)[s]
    r4 = idx - off
    gathered = _make_sc_gather(2 * D, B)(t32, r4)
    return _mlp(gathered, s.reshape(B, 1), W1, b1, W2, b2)


# speedup vs baseline: 1.6522x; 1.6522x over previous
import functools

import jax
import jax.numpy as jnp
from jax import lax
from jax.experimental import pallas as pl
from jax.experimental.pallas import tpu as pltpu
from jax.experimental.pallas import tpu_sc as plsc

_CB = 8192
_N4 = 253952
_OFF = (0, 253952, 507904, 753664)


def _rne16(u):
    return u + 0x7FFF + ((u >> 16) & 1)


def _pack_stream(x_ref):
    x = x_ref[...]
    lo = lax.bitcast_convert_type(x[:32, :], jnp.uint32)
    hi = lax.bitcast_convert_type(x[32:, :], jnp.uint32)
    return (_rne16(lo) >> 16) | (_rne16(hi) & jnp.uint32(0xFFFF0000))


def _repack_body(xa_ref, xb_ref, xc_ref, xd_ref, o_ref):
    w = jnp.concatenate(
        [_pack_stream(r) for r in (xa_ref, xb_ref, xc_ref, xd_ref)], axis=0
    )
    o_ref[...] = lax.bitcast_convert_type(w, jnp.float32).T


def _repack(tableT):
    D, V = tableT.shape
    grid = _N4 // _CB
    return pl.pallas_call(
        _repack_body,
        grid=(grid,),
        in_specs=[
            pl.BlockSpec((D, _CB), lambda i, s=s: (0, i + _OFF[s] // _CB))
            for s in range(4)
        ],
        out_specs=pl.BlockSpec((_CB, 2 * D), lambda i: (i, 0)),
        out_shape=jax.ShapeDtypeStruct((_N4, 2 * D), jnp.float32),
    )(tableT, tableT, tableT, tableT)


def _make_sc_gather(D2, B):
    info = plsc.get_sparse_core_info()
    NC, NS = info.num_cores, info.num_subcores
    NW = NC * NS
    assert B % (8 * NW) == 0 and D2 % info.num_lanes == 0
    b_per_w = B // NW
    mesh = plsc.VectorSubcoreMesh(core_axis_name="c", subcore_axis_name="s")

    @functools.partial(
        pl.kernel,
        mesh=mesh,
        out_type=jax.ShapeDtypeStruct((B, D2), jnp.float32),
        scratch_types=[
            pltpu.VMEM((b_per_w // 128, 128), jnp.int32),
            pltpu.VMEM((b_per_w, D2), jnp.float32),
            pltpu.SemaphoreType.DMA,
            pltpu.SemaphoreType.DMA,
        ],
    )
    def gather_k(table_hbm, idx_hbm, out_hbm, idx_v, rows_v, sem_idx, sem):
        wid = lax.axis_index("s") * NC + lax.axis_index("c")
        base = wid * b_per_w
        nj = b_per_w // 128
        for j in range(nj):
            pltpu.async_copy(
                idx_hbm.at[pl.ds(base + j * 128, 128)], idx_v.at[j], sem_idx
            )
        for j in range(nj):
            pltpu.make_async_copy(
                idx_hbm.at[pl.ds(base + j * 128, 128)], idx_v.at[j], sem_idx
            ).wait()
        for j in range(nj):
            pltpu.async_copy(
                table_hbm.at[idx_v.at[j]], rows_v.at[pl.ds(j * 128, 128)], sem
            )
        for j in range(nj):
            pltpu.make_async_copy(
                table_hbm.at[idx_v.at[j]], rows_v.at[pl.ds(j * 128, 128)], sem
            ).wait()
        pltpu.sync_copy(rows_v, out_hbm.at[pl.ds(base, b_per_w)])

    return gather_k


def _mlp_body(g_ref, sel_ref, w1_ref, b1_ref, w2_ref, b2_ref, o_ref):
    sel = sel_ref[...]
    g = g_ref[...]
    w = jnp.where(
        sel < 2,
        jnp.where(sel == 0, g[:, 0:32], g[:, 32:64]),
        jnp.where(sel == 2, g[:, 64:96], g[:, 96:128]),
    )
    wu = lax.bitcast_convert_type(w, jnp.uint32)
    lo = lax.bitcast_convert_type(wu << 16, jnp.float32)
    hi = lax.bitcast_convert_type(wu & jnp.uint32(0xFFFF0000), jnp.float32)
    x = jnp.concatenate([lo, hi], axis=1)
    h = jnp.dot(x, w1_ref[...], preferred_element_type=jnp.float32) + b1_ref[...]
    h = jnp.maximum(h, 0.0)
    y = jnp.dot(h, w2_ref[...], preferred_element_type=jnp.float32) + b2_ref[...]
    ss = jnp.sum(y * y, axis=-1, keepdims=True)
    o_ref[...] = y / jnp.maximum(jnp.sqrt(ss), 1e-12)


def _mlp(gathered, sel, W1, b1, W2, b2, blk=2048):
    B, D2 = gathered.shape
    D = W1.shape[0]
    H = W1.shape[1]
    O = W2.shape[1]
    return pl.pallas_call(
        _mlp_body,
        grid=(B // blk,),
        in_specs=[
            pl.BlockSpec((blk, D2), lambda i: (i, 0)),
            pl.BlockSpec((blk, 1), lambda i: (i, 0)),
            pl.BlockSpec((D, H), lambda i: (0, 0)),
            pl.BlockSpec((1, H), lambda i: (0, 0)),
            pl.BlockSpec((H, O), lambda i: (0, 0)),
            pl.BlockSpec((1, O), lambda i: (0, 0)),
        ],
        out_specs=pl.BlockSpec((blk, O), lambda i: (i, 0)),
        out_shape=jax.ShapeDtypeStruct((B, O), jnp.float32),
    )(gathered, sel, W1, b1.reshape(1, H), W2, b2.reshape(1, O))


def kernel(indices, table, W1, b1, W2, b2):
    idx = indices.astype(jnp.int32)
    B = idx.shape[0]
    V, D = table.shape
    t32 = _repack(table.T)
    s = (
        (idx >= _OFF[1]).astype(jnp.int32)
        + (idx >= _OFF[2]).astype(jnp.int32)
        + (idx >= _OFF[3]).astype(jnp.int32)
    )
    off = jnp.array(_OFF, dtype=jnp.int32)[s]
    r4 = idx - off
    gathered = _make_sc_gather(2 * D, B)(t32, r4)
    return _mlp(gathered, s.reshape(B, 1), W1, b1, W2, b2)


# speedup vs baseline: 1.6884x; 1.0219x over previous
import functools

import jax
import jax.numpy as jnp
from jax import lax
from jax.experimental import pallas as pl
from jax.experimental.pallas import tpu as pltpu
from jax.experimental.pallas import tpu_sc as plsc

_CB = 8192
_N4 = 253952
_OFF = (0, 253952, 507904, 753664)


def _rne16(u):
    return u + 0x7FFF + ((u >> 16) & 1)


def _pack_stream(x_ref):
    x = x_ref[...]
    lo = lax.bitcast_convert_type(x[:32, :], jnp.uint32)
    hi = lax.bitcast_convert_type(x[32:, :], jnp.uint32)
    return (_rne16(lo) >> 16) | (_rne16(hi) & jnp.uint32(0xFFFF0000))


def _repack_body(xa_ref, xb_ref, xc_ref, xd_ref, o_ref):
    w = jnp.concatenate(
        [_pack_stream(r) for r in (xa_ref, xb_ref, xc_ref, xd_ref)], axis=0
    )
    o_ref[...] = lax.bitcast_convert_type(w, jnp.float32).T


def _repack(tableT):
    D, V = tableT.shape
    grid = _N4 // _CB
    return pl.pallas_call(
        _repack_body,
        grid=(grid,),
        in_specs=[
            pl.BlockSpec((D, _CB), lambda i, s=s: (0, i + _OFF[s] // _CB))
            for s in range(4)
        ],
        out_specs=pl.BlockSpec((_CB, 2 * D), lambda i: (i, 0)),
        out_shape=jax.ShapeDtypeStruct((_N4, 2 * D), jnp.float32),
    )(tableT, tableT, tableT, tableT)


def _make_sc_gather(D2, B):
    info = plsc.get_sparse_core_info()
    NC, NS = info.num_cores, info.num_subcores
    NW = NC * NS
    assert B % (8 * NW) == 0 and D2 % info.num_lanes == 0
    b_per_w = B // NW
    mesh = plsc.VectorSubcoreMesh(core_axis_name="c", subcore_axis_name="s")

    @functools.partial(
        pl.kernel,
        mesh=mesh,
        out_type=jax.ShapeDtypeStruct((B, D2), jnp.float32),
        scratch_types=[
            pltpu.VMEM((b_per_w // 128, 128), jnp.int32),
            pltpu.VMEM((b_per_w, D2), jnp.float32),
            pltpu.SemaphoreType.DMA,
            pltpu.SemaphoreType.DMA,
        ],
    )
    def gather_k(table_hbm, idx_hbm, out_hbm, idx_v, rows_v, sem_idx, sem):
        wid = lax.axis_index("s") * NC + lax.axis_index("c")
        base = wid * b_per_w
        nj = b_per_w // 128
        for j in range(nj):
            pltpu.async_copy(
                idx_hbm.at[pl.ds(base + j * 128, 128)], idx_v.at[j], sem_idx
            )
        for j in range(nj):
            pltpu.make_async_copy(
                idx_hbm.at[pl.ds(base + j * 128, 128)], idx_v.at[j], sem_idx
            ).wait()
        for j in range(nj):
            pltpu.async_copy(
                table_hbm.at[idx_v.at[j]], rows_v.at[pl.ds(j * 128, 128)], sem
            )
        for j in range(nj):
            pltpu.make_async_copy(
                table_hbm.at[idx_v.at[j]], rows_v.at[pl.ds(j * 128, 128)], sem
            ).wait()
        pltpu.sync_copy(rows_v, out_hbm.at[pl.ds(base, b_per_w)])

    return gather_k


def _mlp_body(g_ref, sel_ref, w1_ref, b1_ref, w2_ref, b2_ref, o_ref):
    sel = sel_ref[...]
    g = g_ref[...]
    w = jnp.where(
        sel < 2,
        jnp.where(sel == 0, g[:, 0:32], g[:, 32:64]),
        jnp.where(sel == 2, g[:, 64:96], g[:, 96:128]),
    )
    wu = lax.bitcast_convert_type(w, jnp.uint32)
    lo = lax.bitcast_convert_type(wu << 16, jnp.float32)
    hi = lax.bitcast_convert_type(wu & jnp.uint32(0xFFFF0000), jnp.float32)
    w1 = w1_ref[...]
    h = (
        jnp.dot(lo, w1[:32], preferred_element_type=jnp.float32)
        + jnp.dot(hi, w1[32:], preferred_element_type=jnp.float32)
        + b1_ref[...]
    )
    h = jnp.maximum(h, 0.0)
    y = jnp.dot(h, w2_ref[...], preferred_element_type=jnp.float32) + b2_ref[...]
    ss = jnp.sum(y * y, axis=-1, keepdims=True)
    o_ref[...] = y / jnp.maximum(jnp.sqrt(ss), 1e-12)


def _mlp(gathered, sel, W1, b1, W2, b2, blk=2048):
    B, D2 = gathered.shape
    D = W1.shape[0]
    H = W1.shape[1]
    O = W2.shape[1]
    return pl.pallas_call(
        _mlp_body,
        grid=(B // blk,),
        in_specs=[
            pl.BlockSpec((blk, D2), lambda i: (i, 0)),
            pl.BlockSpec((blk, 1), lambda i: (i, 0)),
            pl.BlockSpec((D, H), lambda i: (0, 0)),
            pl.BlockSpec((1, H), lambda i: (0, 0)),
            pl.BlockSpec((H, O), lambda i: (0, 0)),
            pl.BlockSpec((1, O), lambda i: (0, 0)),
        ],
        out_specs=pl.BlockSpec((blk, O), lambda i: (i, 0)),
        out_shape=jax.ShapeDtypeStruct((B, O), jnp.float32),
    )(gathered, sel, W1, b1.reshape(1, H), W2, b2.reshape(1, O))


def kernel(indices, table, W1, b1, W2, b2):
    idx = indices.astype(jnp.int32)
    B = idx.shape[0]
    V, D = table.shape
    t32 = _repack(table.T)
    s = (
        (idx >= _OFF[1]).astype(jnp.int32)
        + (idx >= _OFF[2]).astype(jnp.int32)
        + (idx >= _OFF[3]).astype(jnp.int32)
    )
    off = jnp.array(_OFF, dtype=jnp.int32)[s]
    r4 = idx - off
    gathered = _make_sc_gather(2 * D, B)(t32, r4)
    return _mlp(gathered, s.reshape(B, 1), W1, b1, W2, b2)


# speedup vs baseline: 1.7090x; 1.0122x over previous
import functools

import jax
import jax.numpy as jnp
from jax import lax
from jax.experimental import pallas as pl
from jax.experimental.pallas import tpu as pltpu
from jax.experimental.pallas import tpu_sc as plsc

_CB = 8192
_N4 = 253952
_OFF = (0, 253952, 507904, 753664)


def _rne16(u):
    return u + 0x7FFF + ((u >> 16) & 1)


def _pack_stream(x_ref):
    x = x_ref[...]
    lo = lax.bitcast_convert_type(x[:32, :], jnp.uint32)
    hi = lax.bitcast_convert_type(x[32:, :], jnp.uint32)
    return (_rne16(lo) >> 16) | (_rne16(hi) & jnp.uint32(0xFFFF0000))


def _repack_body(xa_ref, xb_ref, xc_ref, xd_ref, o_ref):
    w = jnp.concatenate(
        [_pack_stream(r) for r in (xa_ref, xb_ref, xc_ref, xd_ref)], axis=0
    )
    o_ref[...] = lax.bitcast_convert_type(w, jnp.float32).T


def _repack(tableT):
    D, V = tableT.shape
    grid = _N4 // _CB
    return pl.pallas_call(
        _repack_body,
        grid=(grid,),
        in_specs=[
            pl.BlockSpec((D, _CB), lambda i, s=s: (0, i + _OFF[s] // _CB))
            for s in range(4)
        ],
        out_specs=pl.BlockSpec((_CB, 2 * D), lambda i: (i, 0)),
        out_shape=jax.ShapeDtypeStruct((_N4, 2 * D), jnp.float32),
    )(tableT, tableT, tableT, tableT)


def _make_sc_gather(D2, B):
    info = plsc.get_sparse_core_info()
    NC, NS = info.num_cores, info.num_subcores
    NW = NC * NS
    assert B % (8 * NW) == 0 and D2 % info.num_lanes == 0
    b_per_w = B // NW
    mesh = plsc.VectorSubcoreMesh(core_axis_name="c", subcore_axis_name="s")

    @functools.partial(
        pl.kernel,
        mesh=mesh,
        out_type=jax.ShapeDtypeStruct((B, D2), jnp.float32),
        scratch_types=[
            pltpu.VMEM((b_per_w // 128, 128), jnp.int32),
            pltpu.VMEM((b_per_w, D2), jnp.float32),
            pltpu.SemaphoreType.DMA,
            pltpu.SemaphoreType.DMA,
        ],
    )
    def gather_k(table_hbm, idx_hbm, out_hbm, idx_v, rows_v, sem_idx, sem):
        wid = lax.axis_index("s") * NC + lax.axis_index("c")
        base = wid * b_per_w
        nj = b_per_w // 128
        for j in range(nj):
            pltpu.async_copy(
                idx_hbm.at[pl.ds(base + j * 128, 128)], idx_v.at[j], sem_idx
            )
        for j in range(nj):
            pltpu.make_async_copy(
                idx_hbm.at[pl.ds(base + j * 128, 128)], idx_v.at[j], sem_idx
            ).wait()
        for j in range(nj):
            pltpu.async_copy(
                table_hbm.at[idx_v.at[j]], rows_v.at[pl.ds(j * 128, 128)], sem
            )
        for j in range(nj):
            pltpu.make_async_copy(
                table_hbm.at[idx_v.at[j]], rows_v.at[pl.ds(j * 128, 128)], sem
            ).wait()
            pltpu.async_copy(
                rows_v.at[pl.ds(j * 128, 128)],
                out_hbm.at[pl.ds(base + j * 128, 128)],
                sem_idx,
            )
        for j in range(nj):
            pltpu.make_async_copy(
                rows_v.at[pl.ds(j * 128, 128)],
                out_hbm.at[pl.ds(base + j * 128, 128)],
                sem_idx,
            ).wait()

    return gather_k


def _mlp_body(g_ref, sel_ref, w1_ref, b1_ref, w2_ref, b2_ref, o_ref):
    sel = sel_ref[...]
    g = g_ref[...]
    w = jnp.where(
        sel < 2,
        jnp.where(sel == 0, g[:, 0:32], g[:, 32:64]),
        jnp.where(sel == 2, g[:, 64:96], g[:, 96:128]),
    )
    wu = lax.bitcast_convert_type(w, jnp.uint32)
    lo = lax.bitcast_convert_type(wu << 16, jnp.float32)
    hi = lax.bitcast_convert_type(wu & jnp.uint32(0xFFFF0000), jnp.float32)
    w1 = w1_ref[...]
    h = (
        jnp.dot(lo, w1[:32], preferred_element_type=jnp.float32)
        + jnp.dot(hi, w1[32:], preferred_element_type=jnp.float32)
        + b1_ref[...]
    )
    h = jnp.maximum(h, 0.0)
    y = jnp.dot(h, w2_ref[...], preferred_element_type=jnp.float32) + b2_ref[...]
    ss = jnp.sum(y * y, axis=-1, keepdims=True)
    o_ref[...] = y / jnp.maximum(jnp.sqrt(ss), 1e-12)


def _mlp(gathered, sel, W1, b1, W2, b2, blk=4096):
    B, D2 = gathered.shape
    D = W1.shape[0]
    H = W1.shape[1]
    O = W2.shape[1]
    return pl.pallas_call(
        _mlp_body,
        grid=(B // blk,),
        in_specs=[
            pl.BlockSpec((blk, D2), lambda i: (i, 0)),
            pl.BlockSpec((blk, 1), lambda i: (i, 0)),
            pl.BlockSpec((D, H), lambda i: (0, 0)),
            pl.BlockSpec((1, H), lambda i: (0, 0)),
            pl.BlockSpec((H, O), lambda i: (0, 0)),
            pl.BlockSpec((1, O), lambda i: (0, 0)),
        ],
        out_specs=pl.BlockSpec((blk, O), lambda i: (i, 0)),
        out_shape=jax.ShapeDtypeStruct((B, O), jnp.float32),
    )(gathered, sel, W1, b1.reshape(1, H), W2, b2.reshape(1, O))


def kernel(indices, table, W1, b1, W2, b2):
    idx = indices.astype(jnp.int32)
    B = idx.shape[0]
    V, D = table.shape
    t32 = _repack(table.T)
    s = (
        (idx >= _OFF[1]).astype(jnp.int32)
        + (idx >= _OFF[2]).astype(jnp.int32)
        + (idx >= _OFF[3]).astype(jnp.int32)
    )
    off = jnp.array(_OFF, dtype=jnp.int32)[s]
    r4 = idx - off
    gathered = _make_sc_gather(2 * D, B)(t32, r4)
    return _mlp(gathered, s.reshape(B, 1), W1, b1, W2, b2)
